# baseline (speedup 1.0000x reference)
import functools

import jax
import jax.numpy as jnp
from jax import lax
from jax.experimental import pallas as pl
from jax.experimental.pallas import tpu as pltpu
from jax.experimental.pallas import tpu_sc as plsc

BATCH = 16384
FIELDS = 26
EMBED_DIM = 32
B = BATCH * FIELDS
NC, NS = 2, 16
NW = NC * NS
CHUNK = 128
J = B // (NW * CHUNK)
G = 8
M = J // G
PERW = J * CHUNK

_mesh = plsc.VectorSubcoreMesh(core_axis_name="c", subcore_axis_name="s")


def _full(v):
    return jnp.full((16,), v, dtype=jnp.int32)


@functools.partial(
    pl.kernel,
    mesh=_mesh,
    out_type=jax.ShapeDtypeStruct(
        (FIELDS, EMBED_DIM // 8, BATCH // 128, 8, 128), jnp.float32
    ),
    scratch_types=[
        pltpu.VMEM((PERW,), jnp.int32),
        pltpu.VMEM((2, G * CHUNK, EMBED_DIM), jnp.float32),
        pltpu.VMEM((EMBED_DIM // 8, G, 8, 128), jnp.float32),
        pltpu.SemaphoreType.DMA,
        pltpu.SemaphoreType.DMA,
        pltpu.SemaphoreType.DMA,
    ],
    compiler_params=pltpu.CompilerParams(
        use_tc_tiling_on_sc=False, needs_layout_passes=False
    ),
)
def _gather_kernel(idx_hbm, table_hbm, out_hbm, idx_v, buf, tbuf, g0, g1, w0):
    wid = lax.axis_index("s") * NC + lax.axis_index("c")
    pltpu.sync_copy(idx_hbm.at[pl.ds(wid * PERW, PERW)], idx_v)
    gsems = (g0, g1)
    iota = lax.iota(jnp.int32, 16)

    def fire(m, pb):
        for cc in range(G):
            pltpu.async_copy(
                table_hbm.at[idx_v.at[pl.ds((m * G + cc) * CHUNK, CHUNK)]],
                buf.at[pb].at[pl.ds(cc * CHUNK, CHUNK)],
                gsems[pb],
            )

    def drain_g(pb):
        for cc in range(G):
            pltpu.make_async_copy(
                table_hbm.at[pl.ds(0, CHUNK)],
                buf.at[pb].at[pl.ds(cc * CHUNK, CHUNK)],
                gsems[pb],
            ).wait()

    def transpose(pb):
        def body(cc, carry):
            rows = [iota + (cc * CHUNK + p * 16) for p in range(CHUNK // 16)]
            for r in range(EMBED_DIM):
                jb, jm = r // 8, r % 8
                for p in range(CHUNK // 16):
                    vals = plsc.load_gather(buf.at[pb], [rows[p], _full(r)])
                    tbuf[jb, cc, jm, pl.ds(p * 16, 16)] = vals
            return carry

        lax.fori_loop(0, G, body, 0)

    def write(m):
        g_first = wid * J + m * G
        f = g_first // (BATCH // CHUNK)
        bb0 = g_first % (BATCH // CHUNK)
        for jb in range(EMBED_DIM // 8):
            pltpu.async_copy(
                tbuf.at[jb], out_hbm.at[f, jb, pl.ds(bb0, G)], w0
            )

    def wait_w():
        for jb in range(EMBED_DIM // 8):
            pltpu.make_async_copy(
                tbuf.at[jb], out_hbm.at[0, jb, pl.ds(0, G)], w0
            ).wait()

    fire(0, 0)
    fire(1, 1)
    drain_g(0)
    transpose(0)
    write(0)
    fire(2, 0)

    def pair(i, carry):
        m1 = 2 * i + 1
        drain_g(1)
        wait_w()
        transpose(1)
        write(m1)

        @pl.when(i < (M - 3) // 2)
        def _():
            fire(m1 + 2, 1)

        m2 = 2 * i + 2
        drain_g(0)
        wait_w()
        transpose(0)
        write(m2)

        @pl.when(i < (M - 3) // 2)
        def _():
            fire(m2 + 2, 0)

        return carry

    lax.fori_loop(0, (M - 1) // 2, pair, 0)
    wait_w()


def kernel(bin_ids, table):
    idx = jnp.swapaxes(bin_ids, 0, 1).reshape(-1)
    out5 = _gather_kernel(idx, table)
    return out5.transpose(2, 4, 0, 1, 3).reshape(BATCH, FIELDS, EMBED_DIM)


# speedup vs baseline: 1.4408x; 1.4408x over previous
import functools

import jax
import jax.numpy as jnp
from jax import lax
from jax.experimental import pallas as pl
from jax.experimental.pallas import tpu as pltpu
from jax.experimental.pallas import tpu_sc as plsc

BATCH = 16384
FIELDS = 26
EMBED_DIM = 32
B = BATCH * FIELDS
NC, NS = 2, 16
NW = NC * NS
CHUNK = 128
J = B // (NW * CHUNK)
G = 8
M = J // G
PERW = J * CHUNK
GW = G * CHUNK
PITCH = GW + 1

_mesh = plsc.VectorSubcoreMesh(core_axis_name="c", subcore_axis_name="s")


def _full(v):
    return jnp.full((16,), v, dtype=jnp.int32)


@functools.partial(
    pl.kernel,
    mesh=_mesh,
    out_type=jax.ShapeDtypeStruct(
        (FIELDS, EMBED_DIM // 8, BATCH // 128, 8, 128), jnp.float32
    ),
    scratch_types=[
        pltpu.VMEM((PERW,), jnp.int32),
        pltpu.VMEM((2, GW, EMBED_DIM), jnp.float32),
        pltpu.VMEM((EMBED_DIM, PITCH), jnp.float32),
        pltpu.SemaphoreType.DMA,
        pltpu.SemaphoreType.DMA,
        pltpu.SemaphoreType.DMA,
    ],
    compiler_params=pltpu.CompilerParams(
        use_tc_tiling_on_sc=False, needs_layout_passes=False
    ),
)
def _gather_kernel(idx_hbm, table_hbm, out_hbm, idx_v, buf, tbuf, g0, g1, w0):
    wid = lax.axis_index("s") * NC + lax.axis_index("c")
    pltpu.sync_copy(idx_hbm.at[pl.ds(wid * PERW, PERW)], idx_v)
    gsems = (g0, g1)
    iota = lax.iota(jnp.int32, 16)

    def fire(m, pb):
        for cc in range(G):
            pltpu.async_copy(
                table_hbm.at[idx_v.at[pl.ds((m * G + cc) * CHUNK, CHUNK)]],
                buf.at[pb].at[pl.ds(cc * CHUNK, CHUNK)],
                gsems[pb],
            )

    def drain_g(pb):
        for cc in range(G):
            pltpu.make_async_copy(
                table_hbm.at[pl.ds(0, CHUNK)],
                buf.at[pb].at[pl.ds(cc * CHUNK, CHUNK)],
                gsems[pb],
            ).wait()

    def transpose(pb):
        def body(n, carry):
            v0 = buf[pb, n, pl.ds(0, 16)]
            v1 = buf[pb, n, pl.ds(16, 16)]
            plsc.store_scatter(tbuf, [iota, _full(n)], v0)
            plsc.store_scatter(tbuf, [iota + 16, _full(n)], v1)
            return carry

        lax.fori_loop(0, GW, body, 0)

    def write(m):
        g_first = wid * J + m * G
        f = g_first // (BATCH // CHUNK)
        bb0 = g_first % (BATCH // CHUNK)
        for jb in range(EMBED_DIM // 8):
            for cc in range(G):
                pltpu.async_copy(
                    tbuf.at[pl.ds(jb * 8, 8), pl.ds(cc * CHUNK, CHUNK)],
                    out_hbm.at[f, jb, bb0 + cc],
                    w0,
                )

    def wait_w():
        for _ in range(EMBED_DIM // 8 * G):
            pltpu.make_async_copy(
                tbuf.at[pl.ds(0, 8), pl.ds(0, CHUNK)],
                out_hbm.at[0, 0, 0],
                w0,
            ).wait()

    fire(0, 0)
    fire(1, 1)
    drain_g(0)
    transpose(0)
    write(0)
    fire(2, 0)

    def pair(i, carry):
        m1 = 2 * i + 1
        drain_g(1)
        wait_w()
        transpose(1)
        write(m1)

        @pl.when(i < (M - 3) // 2)
        def _():
            fire(m1 + 2, 1)

        m2 = 2 * i + 2
        drain_g(0)
        wait_w()
        transpose(0)
        write(m2)

        @pl.when(i < (M - 3) // 2)
        def _():
            fire(m2 + 2, 0)

        return carry

    lax.fori_loop(0, (M - 1) // 2, pair, 0)
    wait_w()


def kernel(bin_ids, table):
    idx = jnp.swapaxes(bin_ids, 0, 1).reshape(-1)
    out5 = _gather_kernel(idx, table)
    return out5.transpose(2, 4, 0, 1, 3).reshape(BATCH, FIELDS, EMBED_DIM)
